# Initial kernel scaffold; baseline (speedup 1.0000x reference)
import jax
import jax.numpy as jnp
from jax.experimental import pallas as pl


def kernel(x, edge_index, mlp_W0, mlp_b0, mlp_W1, mlp_b1, mlp_W2, mlp_b2, mlp_W3, mlp_b3, mlp_W4, mlp_b4, gcn_W0, gcn_b0, gcn_W1, gcn_b1, gcn_W2, gcn_b2, gcn_W3, gcn_b3, gcn_W4, gcn_b4, out_W, out_b):
    raise NotImplementedError("write your pallas kernel here")


# speedup vs baseline: 8.3433x; 8.3433x over previous
import functools

import jax
import jax.numpy as jnp
from jax import lax
from jax.experimental import pallas as pl
from jax.experimental.pallas import tpu as pltpu
from jax.experimental.pallas import tpu_sc as plsc

N = 10000
NP = 10240
E = 320000
D_IN = 128
H = 256
HD = 128
NC = 2
NS = 16
CH = 128
RPS = NP // NS
DW = 128

_mesh = plsc.VectorSubcoreMesh(
    core_axis_name="c", subcore_axis_name="s", num_cores=NC, num_subcores=NS
)

_E2 = E // NC
_EPW_D = _E2 // NS
_NF_D = _EPW_D // CH
_TL_D = _EPW_D - _NF_D * CH


@functools.partial(
    pl.kernel,
    out_type=jax.ShapeDtypeStruct((NC, NP, DW), jnp.float32),
    mesh=_mesh,
    scratch_types=[
        pltpu.VMEM((CH,), jnp.int32),
        pltpu.VMEM((_TL_D,), jnp.int32),
        pltpu.VMEM((CH, DW), jnp.float32),
        pltpu.VMEM_SHARED((NP, DW), jnp.float32),
    ],
)
def _deg_call(dst, ones_h, zrows, out, di, dit, ones_v, acc):
    c = lax.axis_index("c")
    s = lax.axis_index("s")
    r0 = s * RPS
    pltpu.sync_copy(zrows, acc.at[pl.ds(r0, RPS)])
    pltpu.sync_copy(ones_h, ones_v)
    plsc.subcore_barrier()

    e0 = c * _E2 + s * _EPW_D

    @pl.loop(0, _NF_D)
    def _(j):
        base = e0 + j * CH
        pltpu.sync_copy(dst.at[pl.ds(base, CH)], di)
        pltpu.sync_copy(ones_v, acc.at[di], add=True)

    base = e0 + _NF_D * CH
    pltpu.sync_copy(dst.at[pl.ds(base, _TL_D)], dit)
    pltpu.sync_copy(ones_v.at[pl.ds(0, _TL_D)], acc.at[dit], add=True)

    plsc.subcore_barrier()
    pltpu.sync_copy(acc.at[pl.ds(r0, RPS)], out.at[c, pl.ds(r0, RPS)])


_EPW = E // NS
_NF = _EPW // CH
_TL = _EPW - _NF * CH


@functools.partial(
    pl.kernel,
    out_type=jax.ShapeDtypeStruct((NC, NP, HD), jnp.float32),
    mesh=_mesh,
    scratch_types=[
        pltpu.VMEM((CH,), jnp.int32),
        pltpu.VMEM((CH,), jnp.int32),
        pltpu.VMEM((CH, HD), jnp.float32),
        pltpu.VMEM((_TL,), jnp.int32),
        pltpu.VMEM((_TL,), jnp.int32),
        pltpu.VMEM((_TL, HD), jnp.float32),
        pltpu.VMEM_SHARED((NP, HD), jnp.float32),
    ],
)
def _edge_call(gl, gr, src, dst, zrows, out, si, di, rows, sit, dit, rowst, acc):
    c = lax.axis_index("c")
    s = lax.axis_index("s")
    r0 = s * RPS
    pltpu.sync_copy(zrows, acc.at[pl.ds(r0, RPS)])
    plsc.subcore_barrier()

    e0 = s * _EPW

    def run(table):
        @pl.loop(0, _NF)
        def _(j):
            base = e0 + j * CH
            pltpu.sync_copy(src.at[pl.ds(base, CH)], si)
            pltpu.sync_copy(table.at[si], rows)
            pltpu.sync_copy(dst.at[pl.ds(base, CH)], di)
            pltpu.sync_copy(rows, acc.at[di], add=True)

        base = e0 + _NF * CH
        pltpu.sync_copy(src.at[pl.ds(base, _TL)], sit)
        pltpu.sync_copy(table.at[sit], rowst)
        pltpu.sync_copy(dst.at[pl.ds(base, _TL)], dit)
        pltpu.sync_copy(rowst, acc.at[dit], add=True)

    @pl.when(c == 0)
    def _():
        run(gl)

    @pl.when(c == 1)
    def _():
        run(gr)

    plsc.subcore_barrier()
    pltpu.sync_copy(acc.at[pl.ds(r0, RPS)], out.at[c, pl.ds(r0, RPS)])


BR = 1000


def _mlp_body(x_ref, w0, b0, w1, b1, w2, b2, w3, b3, w4, b4, out_ref):
    h = x_ref[...]
    for w, b in ((w0, b0), (w1, b1), (w2, b2), (w3, b3), (w4, b4)):
        h = jnp.maximum(
            jnp.dot(h, w[...], preferred_element_type=jnp.float32) + b[...], 0.0
        )
    out_ref[...] = h


def _full(shape):
    return pl.BlockSpec(shape, lambda i: (0,) * len(shape))


def _mlp_call(x, ws_bs):
    in_specs = [pl.BlockSpec((BR, D_IN), lambda i: (i, 0))]
    for w, b in ws_bs:
        in_specs += [_full(w.shape), _full(b.shape)]
    flat = [a for wb in ws_bs for a in wb]
    return pl.pallas_call(
        _mlp_body,
        grid=(N // BR,),
        in_specs=in_specs,
        out_specs=pl.BlockSpec((BR, H), lambda i: (i, 0)),
        out_shape=jax.ShapeDtypeStruct((N, H), jnp.float32),
    )(x, *flat)


def _dinv_of(deg_ref):
    d = deg_ref[...]
    return lax.rsqrt(d[0, :, 0:1] + d[1, :, 0:1] + 1.0)


def _pre_body(h_ref, w_ref, deg_ref, gl_ref, gr_ref):
    g = _dinv_of(deg_ref) * jnp.dot(
        h_ref[...], w_ref[...], preferred_element_type=jnp.float32
    )
    gl_ref[...] = g[:, :HD]
    gr_ref[...] = g[:, HD:]


def _pre_call(h, w, deg):
    return pl.pallas_call(
        _pre_body,
        grid=(N // BR,),
        in_specs=[
            pl.BlockSpec((BR, H), lambda i: (i, 0)),
            _full((H, H)),
            pl.BlockSpec((NC, BR, 1), lambda i: (0, i, 0)),
        ],
        out_specs=[
            pl.BlockSpec((BR, HD), lambda i: (i, 0)),
            pl.BlockSpec((BR, HD), lambda i: (i, 0)),
        ],
        out_shape=[
            jax.ShapeDtypeStruct((N, HD), jnp.float32),
            jax.ShapeDtypeStruct((N, HD), jnp.float32),
        ],
    )(h, w, deg)


def _post_body(s2_ref, gl_ref, gr_ref, deg_ref, b_ref, h_ref, out_ref):
    dinv = _dinv_of(deg_ref)
    s2 = s2_ref[...]
    sg = jnp.concatenate([s2[0] + gl_ref[...], s2[1] + gr_ref[...]], axis=-1)
    out_ref[...] = (
        jnp.maximum(dinv * sg + b_ref[...], 0.0) + h_ref[...]
    )


def _post_call(s2, gl, gr, deg, b, h):
    return pl.pallas_call(
        _post_body,
        grid=(N // BR,),
        in_specs=[
            pl.BlockSpec((NC, BR, HD), lambda i: (0, i, 0)),
            pl.BlockSpec((BR, HD), lambda i: (i, 0)),
            pl.BlockSpec((BR, HD), lambda i: (i, 0)),
            pl.BlockSpec((NC, BR, 1), lambda i: (0, i, 0)),
            _full((1, H)),
            pl.BlockSpec((BR, H), lambda i: (i, 0)),
        ],
        out_specs=pl.BlockSpec((BR, H), lambda i: (i, 0)),
        out_shape=jax.ShapeDtypeStruct((N, H), jnp.float32),
    )(s2, gl, gr, deg, b, h)


def _fin_body(h_ref, w_ref, b_ref, out_ref):
    out_ref[...] = (
        jnp.dot(h_ref[...], w_ref[...], preferred_element_type=jnp.float32)
        + b_ref[...]
    )


def _fin_call(h, w, b):
    return pl.pallas_call(
        _fin_body,
        grid=(N // BR,),
        in_specs=[
            pl.BlockSpec((BR, H), lambda i: (i, 0)),
            _full((H, 1)),
            _full((1, 1)),
        ],
        out_specs=pl.BlockSpec((BR, 1), lambda i: (i, 0)),
        out_shape=jax.ShapeDtypeStruct((N, 1), jnp.float32),
    )(h, w, b)


def kernel(x, edge_index,
           mlp_W0, mlp_b0, mlp_W1, mlp_b1, mlp_W2, mlp_b2, mlp_W3, mlp_b3,
           mlp_W4, mlp_b4,
           gcn_W0, gcn_b0, gcn_W1, gcn_b1, gcn_W2, gcn_b2, gcn_W3, gcn_b3,
           gcn_W4, gcn_b4, out_W, out_b):
    src = edge_index[0]
    dst = edge_index[1]

    ones_h = jnp.ones((CH, DW), jnp.float32)
    zrows_d = jnp.zeros((RPS, DW), jnp.float32)
    zrows_e = jnp.zeros((RPS, HD), jnp.float32)

    deg = _deg_call(dst, ones_h, zrows_d)[:, :, :1]

    ws_bs = [
        (mlp_W0, mlp_b0.reshape(1, -1)),
        (mlp_W1, mlp_b1.reshape(1, -1)),
        (mlp_W2, mlp_b2.reshape(1, -1)),
        (mlp_W3, mlp_b3.reshape(1, -1)),
        (mlp_W4, mlp_b4.reshape(1, -1)),
    ]
    h = _mlp_call(x, ws_bs)

    gcn = [
        (gcn_W0, gcn_b0), (gcn_W1, gcn_b1), (gcn_W2, gcn_b2),
        (gcn_W3, gcn_b3), (gcn_W4, gcn_b4),
    ]
    for w, b in gcn:
        gl, gr = _pre_call(h, w, deg)
        s2 = _edge_call(gl, gr, src, dst, zrows_e)
        h = _post_call(s2, gl, gr, deg, b.reshape(1, -1), h)

    preds = _fin_call(h, out_W, out_b.reshape(1, 1))
    return preds[:, 0]


# speedup vs baseline: 16.6505x; 1.9957x over previous
import functools

import jax
import jax.numpy as jnp
from jax import lax
from jax.experimental import pallas as pl
from jax.experimental.pallas import tpu as pltpu
from jax.experimental.pallas import tpu_sc as plsc

N = 10000
NP = 10240
E = 320000
D_IN = 128
H = 256
HD = 128
NC = 2
NS = 16
CH = 128
RPS = NP // NS
DW = 128

_mesh = plsc.VectorSubcoreMesh(
    core_axis_name="c", subcore_axis_name="s", num_cores=NC, num_subcores=NS
)

_NF_D = 80


@functools.partial(
    pl.kernel,
    out_type=jax.ShapeDtypeStruct((NC, NP, DW), jnp.float32),
    mesh=_mesh,
    scratch_types=[
        pltpu.VMEM((_NF_D, CH), jnp.int32),
        pltpu.VMEM((CH, DW), jnp.float32),
        pltpu.VMEM_SHARED((NP, DW), jnp.float32),
    ],
)
def _deg_call(dst2, ones_h, zrows, out, db, ones_v, acc):
    c = lax.axis_index("c")
    s = lax.axis_index("s")
    r0 = s * RPS
    pltpu.sync_copy(zrows, acc.at[pl.ds(r0, RPS)])
    pltpu.sync_copy(ones_h, ones_v)
    w = s * NC + c
    pltpu.sync_copy(dst2.at[pl.ds(w * _NF_D, _NF_D)], db)
    plsc.subcore_barrier()

    @pl.loop(0, _NF_D)
    def _(j):
        pltpu.sync_copy(ones_v, acc.at[db.at[j]], add=True)

    plsc.subcore_barrier()
    pltpu.sync_copy(acc.at[pl.ds(r0, RPS)], out.at[c, pl.ds(r0, RPS)])


KPS = 160
EP = NS * KPS * CH
SG = 32
NST = KPS // SG


@functools.partial(
    pl.kernel,
    out_type=jax.ShapeDtypeStruct((NC, NP, HD), jnp.float32),
    mesh=_mesh,
    scratch_types=[
        pltpu.VMEM((SG, CH), jnp.int32),
        pltpu.VMEM((SG, CH), jnp.int32),
        pltpu.VMEM((CH, HD), jnp.float32),
        pltpu.VMEM((CH, HD), jnp.float32),
        pltpu.VMEM_SHARED((NP, HD), jnp.float32),
        pltpu.SemaphoreType.DMA,
        pltpu.SemaphoreType.DMA,
    ],
)
def _edge_call(gl, gr, src2, dst2, zrows, out, sb, db, rows0, rows1, acc, semA, semB):
    c = lax.axis_index("c")
    s = lax.axis_index("s")
    r0 = s * RPS
    pltpu.sync_copy(zrows, acc.at[pl.ds(r0, RPS)])
    plsc.subcore_barrier()

    def run(table):
        @pl.loop(0, NST)
        def _(t):
            base = s * KPS + t * SG
            pltpu.sync_copy(src2.at[pl.ds(base, SG)], sb)
            pltpu.sync_copy(dst2.at[pl.ds(base, SG)], db)
            pltpu.async_copy(table.at[sb.at[0]], rows0, semA)

            @pl.loop(0, SG // 2 - 1)
            def _(j2):
                j = 2 * j2
                pltpu.async_copy(table.at[sb.at[j + 1]], rows1, semB)
                pltpu.make_async_copy(table.at[sb.at[j]], rows0, semA).wait()
                pltpu.sync_copy(rows0, acc.at[db.at[j]], add=True)
                pltpu.async_copy(table.at[sb.at[j + 2]], rows0, semA)
                pltpu.make_async_copy(table.at[sb.at[j + 1]], rows1, semB).wait()
                pltpu.sync_copy(rows1, acc.at[db.at[j + 1]], add=True)

            j = SG - 2
            pltpu.async_copy(table.at[sb.at[j + 1]], rows1, semB)
            pltpu.make_async_copy(table.at[sb.at[j]], rows0, semA).wait()
            pltpu.sync_copy(rows0, acc.at[db.at[j]], add=True)
            pltpu.make_async_copy(table.at[sb.at[j + 1]], rows1, semB).wait()
            pltpu.sync_copy(rows1, acc.at[db.at[j + 1]], add=True)

    @pl.when(c == 0)
    def _():
        run(gl)

    @pl.when(c == 1)
    def _():
        run(gr)

    plsc.subcore_barrier()
    pltpu.sync_copy(acc.at[pl.ds(r0, RPS)], out.at[c, pl.ds(r0, RPS)])


BR = 1000


def _mlp_body(x_ref, w0, b0, w1, b1, w2, b2, w3, b3, w4, b4, out_ref):
    h = x_ref[...]
    for w, b in ((w0, b0), (w1, b1), (w2, b2), (w3, b3), (w4, b4)):
        h = jnp.maximum(
            jnp.dot(h, w[...], preferred_element_type=jnp.float32) + b[...], 0.0
        )
    out_ref[...] = h


def _full(shape):
    return pl.BlockSpec(shape, lambda i: (0,) * len(shape))


def _mlp_call(x, ws_bs):
    in_specs = [pl.BlockSpec((BR, D_IN), lambda i: (i, 0))]
    for w, b in ws_bs:
        in_specs += [_full(w.shape), _full(b.shape)]
    flat = [a for wb in ws_bs for a in wb]
    return pl.pallas_call(
        _mlp_body,
        grid=(N // BR,),
        in_specs=in_specs,
        out_specs=pl.BlockSpec((BR, H), lambda i: (i, 0)),
        out_shape=jax.ShapeDtypeStruct((N, H), jnp.float32),
    )(x, *flat)


def _dinv_of(deg_ref):
    d = deg_ref[...]
    return lax.rsqrt(d[0, :, 0:1] + d[1, :, 0:1] + 1.0)


def _pre_body(h_ref, w_ref, deg_ref, gl_ref, gr_ref):
    g = _dinv_of(deg_ref) * jnp.dot(
        h_ref[...], w_ref[...], preferred_element_type=jnp.float32
    )
    gl_ref[...] = g[:, :HD]
    gr_ref[...] = g[:, HD:]


def _pre_call(h, w, deg):
    return pl.pallas_call(
        _pre_body,
        grid=(N // BR,),
        in_specs=[
            pl.BlockSpec((BR, H), lambda i: (i, 0)),
            _full((H, H)),
            pl.BlockSpec((NC, BR, 1), lambda i: (0, i, 0)),
        ],
        out_specs=[
            pl.BlockSpec((BR, HD), lambda i: (i, 0)),
            pl.BlockSpec((BR, HD), lambda i: (i, 0)),
        ],
        out_shape=[
            jax.ShapeDtypeStruct((N, HD), jnp.float32),
            jax.ShapeDtypeStruct((N, HD), jnp.float32),
        ],
    )(h, w, deg)


def _post_body(s2_ref, gl_ref, gr_ref, deg_ref, b_ref, h_ref, out_ref):
    dinv = _dinv_of(deg_ref)
    s2 = s2_ref[...]
    sg = jnp.concatenate([s2[0] + gl_ref[...], s2[1] + gr_ref[...]], axis=-1)
    out_ref[...] = (
        jnp.maximum(dinv * sg + b_ref[...], 0.0) + h_ref[...]
    )


def _post_call(s2, gl, gr, deg, b, h):
    return pl.pallas_call(
        _post_body,
        grid=(N // BR,),
        in_specs=[
            pl.BlockSpec((NC, BR, HD), lambda i: (0, i, 0)),
            pl.BlockSpec((BR, HD), lambda i: (i, 0)),
            pl.BlockSpec((BR, HD), lambda i: (i, 0)),
            pl.BlockSpec((NC, BR, 1), lambda i: (0, i, 0)),
            _full((1, H)),
            pl.BlockSpec((BR, H), lambda i: (i, 0)),
        ],
        out_specs=pl.BlockSpec((BR, H), lambda i: (i, 0)),
        out_shape=jax.ShapeDtypeStruct((N, H), jnp.float32),
    )(s2, gl, gr, deg, b, h)


def _fin_body(h_ref, w_ref, b_ref, out_ref):
    out_ref[...] = (
        jnp.dot(h_ref[...], w_ref[...], preferred_element_type=jnp.float32)
        + b_ref[...]
    )


def _fin_call(h, w, b):
    return pl.pallas_call(
        _fin_body,
        grid=(N // BR,),
        in_specs=[
            pl.BlockSpec((BR, H), lambda i: (i, 0)),
            _full((H, 1)),
            _full((1, 1)),
        ],
        out_specs=pl.BlockSpec((BR, 1), lambda i: (i, 0)),
        out_shape=jax.ShapeDtypeStruct((N, 1), jnp.float32),
    )(h, w, b)


def kernel(x, edge_index,
           mlp_W0, mlp_b0, mlp_W1, mlp_b1, mlp_W2, mlp_b2, mlp_W3, mlp_b3,
           mlp_W4, mlp_b4,
           gcn_W0, gcn_b0, gcn_W1, gcn_b1, gcn_W2, gcn_b2, gcn_W3, gcn_b3,
           gcn_W4, gcn_b4, out_W, out_b):
    src = edge_index[0]
    dst = edge_index[1]

    npad = EP - E
    pad_ar = jnp.arange(npad, dtype=jnp.int32)
    src2 = jnp.concatenate([src, pad_ar % N]).reshape(EP // CH, CH)
    dst2 = jnp.concatenate([dst, N + pad_ar % (NP - N)]).reshape(EP // CH, CH)

    ones_h = jnp.ones((CH, DW), jnp.float32)
    zrows_d = jnp.zeros((RPS, DW), jnp.float32)
    zrows_e = jnp.zeros((RPS, HD), jnp.float32)

    deg = _deg_call(dst2, ones_h, zrows_d)[:, :, :1]

    ws_bs = [
        (mlp_W0, mlp_b0.reshape(1, -1)),
        (mlp_W1, mlp_b1.reshape(1, -1)),
        (mlp_W2, mlp_b2.reshape(1, -1)),
        (mlp_W3, mlp_b3.reshape(1, -1)),
        (mlp_W4, mlp_b4.reshape(1, -1)),
    ]
    h = _mlp_call(x, ws_bs)

    gcn = [
        (gcn_W0, gcn_b0), (gcn_W1, gcn_b1), (gcn_W2, gcn_b2),
        (gcn_W3, gcn_b3), (gcn_W4, gcn_b4),
    ]
    for w, b in gcn:
        gl, gr = _pre_call(h, w, deg)
        s2 = _edge_call(gl, gr, src2, dst2, zrows_e)
        h = _post_call(s2, gl, gr, deg, b.reshape(1, -1), h)

    preds = _fin_call(h, out_W, out_b.reshape(1, 1))
    return preds[:, 0]
